# baseline (speedup 1.0000x reference)
import functools

import jax
import jax.numpy as jnp
from jax import lax
from jax.experimental import pallas as pl
from jax.experimental.pallas import tpu as pltpu
from jax.experimental.pallas import tpu_sc as plsc

_F = 26
_B = 16384
_D = 32
_VF = 100000
_V = _F * _VF
_W = 100224
_A0MAX = _V - _W
_NC = 2
_CH = 2048
_NCH = _B // _CH


def _sc_body(xt_hbm, tt_hbm, out_hbm, win_v, idx_v, out_v):
    d = lax.axis_index("s") * _NC + lax.axis_index("c")

    def do_field(f, carry):
        a0 = jnp.minimum(f * _VF // 128 * 128, _A0MAX)
        off = f * _VF - a0
        pltpu.sync_copy(tt_hbm.at[d, pl.ds(a0, _W)], win_v)

        def do_chunk(c, carry2):
            b0 = c * _CH
            pltpu.sync_copy(xt_hbm.at[f, pl.ds(b0, _CH)], idx_v)

            def g16(i, _):
                iv = idx_v[pl.ds(i * 16, 16)] + off
                out_v[pl.ds(i * 16, 16)] = plsc.load_gather(win_v, [iv])
                return 0

            lax.fori_loop(0, _CH // 16, g16, 0)
            pltpu.sync_copy(out_v, out_hbm.at[f, d, pl.ds(b0, _CH)])
            return carry2

        lax.fori_loop(0, _NCH, do_chunk, 0)
        return carry

    lax.fori_loop(0, _F, do_field, 0)


@jax.jit
def _sc_embed(xt, tt):
    mesh = plsc.VectorSubcoreMesh(core_axis_name="c", subcore_axis_name="s")
    run = functools.partial(
        pl.kernel,
        mesh=mesh,
        out_type=jax.ShapeDtypeStruct((_F, _D, _B), jnp.float32),
        scratch_types=[
            pltpu.VMEM((_W,), jnp.float32),
            pltpu.VMEM((_CH,), jnp.int32),
            pltpu.VMEM((_CH,), jnp.float32),
        ],
        compiler_params=pltpu.CompilerParams(
            use_tc_tiling_on_sc=False, needs_layout_passes=False
        ),
    )(_sc_body)
    return run(xt, tt)


def kernel(input_x, table):
    out = _sc_embed(input_x.T, table.T)
    return out.transpose(2, 0, 1)


# speedup vs baseline: 1.2856x; 1.2856x over previous
import functools

import jax
import jax.numpy as jnp
from jax import lax
from jax.experimental import pallas as pl
from jax.experimental.pallas import tpu as pltpu
from jax.experimental.pallas import tpu_sc as plsc

_F = 26
_B = 16384
_D = 32
_VF = 100000
_V = _F * _VF
_NJ = 20320
_WJ = 783
_NC = 2
_CH = 2048
_NCH = _B // _CH


def _retile_body(in_ref, out_ref):
    out_ref[...] = in_ref[...].reshape(8, 8, 128)


@jax.jit
def _tc_retile(tt):
    return pl.pallas_call(
        _retile_body,
        grid=(4, _NJ // 8),
        in_specs=[pl.BlockSpec((8, 1024), lambda i, j: (i, j))],
        out_specs=pl.BlockSpec((8, 8, 128), lambda i, j: (i, j, 0)),
        out_shape=jax.ShapeDtypeStruct((_D, _NJ, 128), jnp.float32),
    )(tt)


def _sc_body(xt_hbm, t3d_hbm, out_hbm, win_v, idx_v, out_v):
    d = lax.axis_index("s") * _NC + lax.axis_index("c")

    def do_field(f, carry):
        j0 = f * _VF // 128
        off = f * _VF - j0 * 128
        pltpu.sync_copy(t3d_hbm.at[d, pl.ds(j0, _WJ)], win_v)

        def do_chunk(c, carry2):
            b0 = c * _CH
            pltpu.sync_copy(xt_hbm.at[f, pl.ds(b0, _CH)], idx_v)

            def g16(i, _):
                iv = idx_v[pl.ds(i * 16, 16)] + off
                out_v[pl.ds(i * 16, 16)] = plsc.load_gather(
                    win_v, [iv >> 7, iv & 127]
                )
                return 0

            lax.fori_loop(0, _CH // 16, g16, 0)
            pltpu.sync_copy(out_v, out_hbm.at[f, d, pl.ds(b0, _CH)])
            return carry2

        lax.fori_loop(0, _NCH, do_chunk, 0)
        return carry

    lax.fori_loop(0, _F, do_field, 0)


@jax.jit
def _sc_embed(xt, t3d):
    mesh = plsc.VectorSubcoreMesh(core_axis_name="c", subcore_axis_name="s")
    run = functools.partial(
        pl.kernel,
        mesh=mesh,
        out_type=jax.ShapeDtypeStruct((_F, _D, _B), jnp.float32),
        scratch_types=[
            pltpu.VMEM((_WJ, 128), jnp.float32),
            pltpu.VMEM((_CH,), jnp.int32),
            pltpu.VMEM((_CH,), jnp.float32),
        ],
        compiler_params=pltpu.CompilerParams(
            use_tc_tiling_on_sc=False, needs_layout_passes=False
        ),
    )(_sc_body)
    return run(xt, t3d)


def kernel(input_x, table):
    t3d = _tc_retile(table.T)
    out = _sc_embed(input_x.T, t3d)
    return out.transpose(2, 0, 1)


# speedup vs baseline: 5.7185x; 4.4483x over previous
import functools

import jax
import jax.numpy as jnp
from jax import lax
from jax.experimental import pallas as pl
from jax.experimental.pallas import tpu as pltpu
from jax.experimental.pallas import tpu_sc as plsc

_F = 26
_B = 16384
_D = 32
_VF = 100000
_NJ = 20320
_WJ = 783
_NC = 2
_CH = 2048
_NCH = _B // _CH
_KJ = 80


def _retile_body(in_ref, out_ref):
    for k in range(_KJ):
        out_ref[0, k] = in_ref[:, 128 * k:128 * (k + 1)]


@jax.jit
def _tc_retile(tt):
    return pl.pallas_call(
        _retile_body,
        grid=(4, _NJ // _KJ),
        in_specs=[pl.BlockSpec((8, 128 * _KJ), lambda i, j: (i, j))],
        out_specs=pl.BlockSpec((1, _KJ, 8, 128), lambda i, j: (i, j, 0, 0)),
        out_shape=jax.ShapeDtypeStruct((4, _NJ, 8, 128), jnp.float32),
    )(tt)


def _sc_body(xt_hbm, t4_hbm, out_hbm, win_v, idx_v, out_v):
    w = lax.axis_index("s") * _NC + lax.axis_index("c")
    g = w // 8
    r = w % 8

    def do_field(f, carry):
        j0 = f * _VF // 128
        off = f * _VF - j0 * 128
        pltpu.sync_copy(t4_hbm.at[g, pl.ds(j0, _WJ), r], win_v)

        def do_chunk(c, carry2):
            b0 = c * _CH
            pltpu.sync_copy(xt_hbm.at[f, pl.ds(b0, _CH)], idx_v)

            def g16(i, _):
                iv = idx_v[pl.ds(i * 16, 16)] + off
                out_v[i // 8, pl.ds((i % 8) * 16, 16)] = plsc.load_gather(
                    win_v, [iv >> 7, iv & 127]
                )
                return 0

            lax.fori_loop(0, _CH // 16, g16, 0)
            pltpu.sync_copy(
                out_v, out_hbm.at[f, g, pl.ds(b0 // 128, _CH // 128), r]
            )
            return carry2

        lax.fori_loop(0, _NCH, do_chunk, 0)
        return carry

    lax.fori_loop(0, _F, do_field, 0)


@jax.jit
def _sc_embed(xt, t4):
    mesh = plsc.VectorSubcoreMesh(core_axis_name="c", subcore_axis_name="s")
    run = functools.partial(
        pl.kernel,
        mesh=mesh,
        out_type=jax.ShapeDtypeStruct((_F, 4, _B // 128, 8, 128), jnp.float32),
        scratch_types=[
            pltpu.VMEM((_WJ, 128), jnp.float32),
            pltpu.VMEM((_CH,), jnp.int32),
            pltpu.VMEM((_CH // 128, 128), jnp.float32),
        ],
        compiler_params=pltpu.CompilerParams(
            use_tc_tiling_on_sc=False, needs_layout_passes=False
        ),
    )(_sc_body)
    return run(xt, t4)


def kernel(input_x, table):
    t4 = _tc_retile(table.T)
    out5 = _sc_embed(input_x.T, t4)
    out = out5.transpose(0, 1, 3, 2, 4).reshape(_F, _D, _B)
    return out.transpose(2, 0, 1)


# speedup vs baseline: 6.9412x; 1.2138x over previous
import functools

import jax
import jax.numpy as jnp
from jax import lax
from jax.experimental import pallas as pl
from jax.experimental.pallas import tpu as pltpu
from jax.experimental.pallas import tpu_sc as plsc

_F = 26
_B = 16384
_D = 32
_VF = 100000
_WJ = 783
_NC = 2
_CH = 2048
_NCH = _B // _CH
_KJ = 80

_SLICES = (
    (0, 13, 0, 10160),
    (13, 13, 10080, 10240),
)


def _retile_body(in_ref, out_ref):
    for k in range(_KJ):
        out_ref[0, k] = in_ref[:, 128 * k:128 * (k + 1)]


@functools.partial(jax.jit, static_argnums=(1, 2))
def _tc_retile(tt, j_base, ng):
    jb = j_base // _KJ
    return pl.pallas_call(
        _retile_body,
        grid=(4, ng // _KJ),
        in_specs=[pl.BlockSpec((8, 128 * _KJ), lambda i, j: (i, j + jb))],
        out_specs=pl.BlockSpec((1, _KJ, 8, 128), lambda i, j: (i, j, 0, 0)),
        out_shape=jax.ShapeDtypeStruct((4, ng, 8, 128), jnp.float32),
    )(tt)


def _sc_body(f0, nf, j_base, xt_hbm, t4_hbm, out_hbm, win_v, idx_v, out_v):
    w = lax.axis_index("s") * _NC + lax.axis_index("c")
    g = w // 8
    r = w % 8

    def do_field(f, carry):
        fg = f0 + f
        j0 = fg * _VF // 128
        off = fg * _VF - j0 * 128
        pltpu.sync_copy(t4_hbm.at[g, pl.ds(j0 - j_base, _WJ), r], win_v)

        def do_chunk(c, carry2):
            b0 = c * _CH
            pltpu.sync_copy(xt_hbm.at[fg, pl.ds(b0, _CH)], idx_v)

            def g16(i, _):
                iv = idx_v[pl.ds(i * 16, 16)] + off
                out_v[i // 8, pl.ds((i % 8) * 16, 16)] = plsc.load_gather(
                    win_v, [iv >> 7, iv & 127]
                )
                return 0

            lax.fori_loop(0, _CH // 16, g16, 0)
            pltpu.sync_copy(
                out_v, out_hbm.at[f, g, pl.ds(b0 // 128, _CH // 128), r]
            )
            return carry2

        lax.fori_loop(0, _NCH, do_chunk, 0)
        return carry

    lax.fori_loop(0, nf, do_field, 0)


@functools.partial(jax.jit, static_argnums=(2, 3, 4))
def _sc_embed(xt, t4, f0, nf, j_base):
    mesh = plsc.VectorSubcoreMesh(core_axis_name="c", subcore_axis_name="s")
    run = functools.partial(
        pl.kernel,
        mesh=mesh,
        out_type=jax.ShapeDtypeStruct((nf, 4, _B // 128, 8, 128), jnp.float32),
        scratch_types=[
            pltpu.VMEM((_WJ, 128), jnp.float32),
            pltpu.VMEM((_CH,), jnp.int32),
            pltpu.VMEM((_CH // 128, 128), jnp.float32),
        ],
        compiler_params=pltpu.CompilerParams(
            use_tc_tiling_on_sc=False, needs_layout_passes=False
        ),
    )(functools.partial(_sc_body, f0, nf, j_base))
    return run(xt, t4)


def kernel(input_x, table):
    tt = table.T
    xt = input_x.T
    outs = []
    for f0, nf, j_base, ng in _SLICES:
        t4 = _tc_retile(tt, j_base, ng)
        outs.append(_sc_embed(xt, t4, f0, nf, j_base))
    out5 = jnp.concatenate(outs, axis=0)
    out = out5.transpose(0, 1, 3, 2, 4).reshape(_F, _D, _B)
    return out.transpose(2, 0, 1)


# speedup vs baseline: 7.3490x; 1.0588x over previous
import functools

import jax
import jax.numpy as jnp
from jax import lax
from jax.experimental import pallas as pl
from jax.experimental.pallas import tpu as pltpu
from jax.experimental.pallas import tpu_sc as plsc

_F = 26
_B = 16384
_D = 32
_VF = 100000
_WJ = 783
_NC = 2
_CH = 2048
_NCH = _B // _CH
_KJ = 80

_SLICES = (
    (0, 3, 0, 2400),
    (3, 10, 2320, 7840),
    (13, 10, 10080, 7920),
    (23, 3, 17920, 2400),
)


def _retile_body(in_ref, out_ref):
    for k in range(_KJ):
        out_ref[0, k] = in_ref[:, 128 * k:128 * (k + 1)]


@functools.partial(jax.jit, static_argnums=(1, 2))
def _tc_retile(tt, j_base, ng):
    jb = j_base // _KJ
    return pl.pallas_call(
        _retile_body,
        grid=(4, ng // _KJ),
        in_specs=[pl.BlockSpec((8, 128 * _KJ), lambda i, j: (i, j + jb))],
        out_specs=pl.BlockSpec((1, _KJ, 8, 128), lambda i, j: (i, j, 0, 0)),
        out_shape=jax.ShapeDtypeStruct((4, ng, 8, 128), jnp.float32),
    )(tt)


def _sc_body(f0, nf, j_base, xt_hbm, t4_hbm, out_hbm, win_v, idx_v, out_v):
    w = lax.axis_index("s") * _NC + lax.axis_index("c")
    g = w // 8
    r = w % 8

    def do_field(f, carry):
        fg = f0 + f
        j0 = fg * _VF // 128
        off = fg * _VF - j0 * 128
        pltpu.sync_copy(t4_hbm.at[g, pl.ds(j0 - j_base, _WJ), r], win_v)

        def do_chunk(c, carry2):
            b0 = c * _CH
            pltpu.sync_copy(xt_hbm.at[fg, pl.ds(b0, _CH)], idx_v)

            def g16(i, _):
                iv = idx_v[pl.ds(i * 16, 16)] + off
                out_v[i // 8, pl.ds((i % 8) * 16, 16)] = plsc.load_gather(
                    win_v, [iv >> 7, iv & 127]
                )
                return 0

            lax.fori_loop(0, _CH // 16, g16, 0)
            pltpu.sync_copy(
                out_v, out_hbm.at[f, g, pl.ds(b0 // 128, _CH // 128), r]
            )
            return carry2

        lax.fori_loop(0, _NCH, do_chunk, 0)
        return carry

    lax.fori_loop(0, nf, do_field, 0)


@functools.partial(jax.jit, static_argnums=(2, 3, 4))
def _sc_embed(xt, t4, f0, nf, j_base):
    mesh = plsc.VectorSubcoreMesh(core_axis_name="c", subcore_axis_name="s")
    run = functools.partial(
        pl.kernel,
        mesh=mesh,
        out_type=jax.ShapeDtypeStruct((nf, 4, _B // 128, 8, 128), jnp.float32),
        scratch_types=[
            pltpu.VMEM((_WJ, 128), jnp.float32),
            pltpu.VMEM((_CH,), jnp.int32),
            pltpu.VMEM((_CH // 128, 128), jnp.float32),
        ],
        compiler_params=pltpu.CompilerParams(
            use_tc_tiling_on_sc=False, needs_layout_passes=False
        ),
    )(functools.partial(_sc_body, f0, nf, j_base))
    return run(xt, t4)


def kernel(input_x, table):
    tt = table.T
    xt = input_x.T
    outs = []
    for f0, nf, j_base, ng in _SLICES:
        t4 = _tc_retile(tt, j_base, ng)
        outs.append(_sc_embed(xt, t4, f0, nf, j_base))
    out5 = jnp.concatenate(outs, axis=0)
    out = out5.transpose(0, 1, 3, 2, 4).reshape(_F, _D, _B)
    return out.transpose(2, 0, 1)


# speedup vs baseline: 8.0901x; 1.1008x over previous
import functools

import jax
import jax.numpy as jnp
from jax import lax
from jax.experimental import pallas as pl
from jax.experimental.pallas import tpu as pltpu
from jax.experimental.pallas import tpu_sc as plsc

_F = 26
_B = 16384
_D = 32
_VF = 100000
_WJ = 783
_NC = 2
_CH = 2048
_NCH = _B // _CH
_KJ = 80

_SLICES = (
    (0, 5, 0, 3920),
    (5, 5, 3840, 4000),
    (10, 6, 7760, 4800),
    (16, 5, 12480, 4000),
    (21, 5, 16400, 3920),
)


def _retile_body(in_ref, out_ref):
    for k in range(_KJ):
        out_ref[0, k] = in_ref[:, 128 * k:128 * (k + 1)]


@functools.partial(jax.jit, static_argnums=(1, 2))
def _tc_retile(tt, j_base, ng):
    jb = j_base // _KJ
    return pl.pallas_call(
        _retile_body,
        grid=(4, ng // _KJ),
        in_specs=[pl.BlockSpec((8, 128 * _KJ), lambda i, j: (i, j + jb))],
        out_specs=pl.BlockSpec((1, _KJ, 8, 128), lambda i, j: (i, j, 0, 0)),
        out_shape=jax.ShapeDtypeStruct((4, ng, 8, 128), jnp.float32),
    )(tt)


def _sc_body(f0, nf, j_base, xt_hbm, t4_hbm, out_hbm, win_v, idx_v, out_v):
    w = lax.axis_index("s") * _NC + lax.axis_index("c")
    g = w // 8
    r = w % 8

    def do_field(f, carry):
        fg = f0 + f
        j0 = fg * _VF // 128
        off = fg * _VF - j0 * 128
        pltpu.sync_copy(t4_hbm.at[g, pl.ds(j0 - j_base, _WJ), r], win_v)

        def do_chunk(c, carry2):
            b0 = c * _CH
            pltpu.sync_copy(xt_hbm.at[fg, pl.ds(b0, _CH)], idx_v)

            def g16(i, _):
                iv = idx_v[pl.ds(i * 16, 16)] + off
                out_v[i // 8, pl.ds((i % 8) * 16, 16)] = plsc.load_gather(
                    win_v, [iv >> 7, iv & 127]
                )
                return 0

            lax.fori_loop(0, _CH // 16, g16, 0)
            pltpu.sync_copy(
                out_v, out_hbm.at[f, g, pl.ds(b0 // 128, _CH // 128), r]
            )
            return carry2

        lax.fori_loop(0, _NCH, do_chunk, 0)
        return carry

    lax.fori_loop(0, nf, do_field, 0)


@functools.partial(jax.jit, static_argnums=(2, 3, 4))
def _sc_embed(xt, t4, f0, nf, j_base):
    mesh = plsc.VectorSubcoreMesh(core_axis_name="c", subcore_axis_name="s")
    run = functools.partial(
        pl.kernel,
        mesh=mesh,
        out_type=jax.ShapeDtypeStruct((nf, 4, _B // 128, 8, 128), jnp.float32),
        scratch_types=[
            pltpu.VMEM((_WJ, 128), jnp.float32),
            pltpu.VMEM((_CH,), jnp.int32),
            pltpu.VMEM((_CH // 128, 128), jnp.float32),
        ],
        compiler_params=pltpu.CompilerParams(
            use_tc_tiling_on_sc=False, needs_layout_passes=False
        ),
    )(functools.partial(_sc_body, f0, nf, j_base))
    return run(xt, t4)


def kernel(input_x, table):
    tt = table.T
    xt = input_x.T
    outs = []
    for f0, nf, j_base, ng in _SLICES:
        t4 = _tc_retile(tt, j_base, ng)
        outs.append(_sc_embed(xt, t4, f0, nf, j_base))
    out5 = jnp.concatenate(outs, axis=0)
    out = out5.transpose(0, 1, 3, 2, 4).reshape(_F, _D, _B)
    return out.transpose(2, 0, 1)


# speedup vs baseline: 8.2685x; 1.0220x over previous
import functools

import jax
import jax.numpy as jnp
from jax import lax
from jax.experimental import pallas as pl
from jax.experimental.pallas import tpu as pltpu
from jax.experimental.pallas import tpu_sc as plsc

_F = 26
_B = 16384
_D = 32
_VF = 100000
_WJ = 783
_NC = 2
_CH = 2048
_NCH = _B // _CH
_KJ = 80

_SLICES = (
    (0, 4, 0, 3200),
    (4, 4, 3120, 3200),
    (8, 4, 6240, 3200),
    (12, 4, 9360, 3200),
    (16, 4, 12480, 3200),
    (20, 3, 15600, 2400),
    (23, 3, 17920, 2400),
)


def _retile_body(in_ref, out_ref):
    for k in range(_KJ):
        out_ref[0, k] = in_ref[:, 128 * k:128 * (k + 1)]


@functools.partial(jax.jit, static_argnums=(1, 2))
def _tc_retile(tt, j_base, ng):
    jb = j_base // _KJ
    return pl.pallas_call(
        _retile_body,
        grid=(4, ng // _KJ),
        in_specs=[pl.BlockSpec((8, 128 * _KJ), lambda i, j: (i, j + jb))],
        out_specs=pl.BlockSpec((1, _KJ, 8, 128), lambda i, j: (i, j, 0, 0)),
        out_shape=jax.ShapeDtypeStruct((4, ng, 8, 128), jnp.float32),
    )(tt)


def _sc_body(f0, nf, j_base, xt_hbm, t4_hbm, out_hbm, win_v, idx_v, out_v):
    w = lax.axis_index("s") * _NC + lax.axis_index("c")
    g = w // 8
    r = w % 8

    def do_field(f, carry):
        fg = f0 + f
        j0 = fg * _VF // 128
        off = fg * _VF - j0 * 128
        pltpu.sync_copy(t4_hbm.at[g, pl.ds(j0 - j_base, _WJ), r], win_v)

        def do_chunk(c, carry2):
            b0 = c * _CH
            pltpu.sync_copy(xt_hbm.at[fg, pl.ds(b0, _CH)], idx_v)

            def g16(i, _):
                iv = idx_v[pl.ds(i * 16, 16)] + off
                out_v[i // 8, pl.ds((i % 8) * 16, 16)] = plsc.load_gather(
                    win_v, [iv >> 7, iv & 127]
                )
                return 0

            lax.fori_loop(0, _CH // 16, g16, 0)
            pltpu.sync_copy(
                out_v, out_hbm.at[f, g, pl.ds(b0 // 128, _CH // 128), r]
            )
            return carry2

        lax.fori_loop(0, _NCH, do_chunk, 0)
        return carry

    lax.fori_loop(0, nf, do_field, 0)


@functools.partial(jax.jit, static_argnums=(2, 3, 4))
def _sc_embed(xt, t4, f0, nf, j_base):
    mesh = plsc.VectorSubcoreMesh(core_axis_name="c", subcore_axis_name="s")
    run = functools.partial(
        pl.kernel,
        mesh=mesh,
        out_type=jax.ShapeDtypeStruct((nf, 4, _B // 128, 8, 128), jnp.float32),
        scratch_types=[
            pltpu.VMEM((_WJ, 128), jnp.float32),
            pltpu.VMEM((_CH,), jnp.int32),
            pltpu.VMEM((_CH // 128, 128), jnp.float32),
        ],
        compiler_params=pltpu.CompilerParams(
            use_tc_tiling_on_sc=False, needs_layout_passes=False
        ),
    )(functools.partial(_sc_body, f0, nf, j_base))
    return run(xt, t4)


def kernel(input_x, table):
    tt = table.T
    xt = input_x.T
    outs = []
    for f0, nf, j_base, ng in _SLICES:
        t4 = _tc_retile(tt, j_base, ng)
        outs.append(_sc_embed(xt, t4, f0, nf, j_base))
    out5 = jnp.concatenate(outs, axis=0)
    out = out5.transpose(0, 1, 3, 2, 4).reshape(_F, _D, _B)
    return out.transpose(2, 0, 1)
